# Initial kernel scaffold; baseline (speedup 1.0000x reference)
import functools

import jax
import jax.numpy as jnp
from jax import lax
from jax.experimental import pallas as pl
from jax.experimental.pallas import tpu as pltpu


def _mask_body(conf_ref, fcw_ref, fcb_ref, lam_ref, mask_ref, *, N, B, H, W, K):
    Vb = N // B
    conf = conf_ref[...]
    cm = jax.nn.sigmoid(conf).max(axis=1)
    lam = lam_ref[0]
    fcw = fcw_ref[...]
    fcb = fcb_ref[...]

    tf = (
        jnp.dot(cm.reshape(N * H, W), fcw.T, preferred_element_type=jnp.float32)
        + fcb[None, :]
    ).reshape(N, H, H)

    comms = []
    for v in range(N):
        ego = (v // Vb) * Vb
        req = 1.0 - cm[ego]
        diff = cm[v] * req
        sim = jnp.dot(tf[v], cm[v], preferred_element_type=jnp.float32)
        comms.append(lam * diff + (1.0 - lam) * sim)
    comm = jnp.stack(comms, axis=0)

    ki = lax.bitcast_convert_type(comm, jnp.int32)
    k3 = jnp.where(ki < 0, ki ^ jnp.int32(0x7FFFFFFF), ki)

    def t_step(i, tu):
        bit = lax.shift_left(jnp.int32(1), jnp.int32(31) - i)
        cand_u = tu | bit
        cand_s = cand_u ^ jnp.int32(-2147483648)
        cnt = jnp.sum((k3 >= cand_s[:, None, None]).astype(jnp.int32), axis=(1, 2))
        return jnp.where(cnt >= K, cand_u, tu)

    tu = lax.fori_loop(0, 32, t_step, jnp.zeros((N,), jnp.int32))
    ts = tu ^ jnp.int32(-2147483648)

    gt = k3 > ts[:, None, None]
    eq = k3 == ts[:, None, None]
    need = K - jnp.sum(gt.astype(jnp.int32), axis=(1, 2))

    ih = lax.broadcasted_iota(jnp.int32, (N, H, W), 1)
    iw = lax.broadcasted_iota(jnp.int32, (N, H, W), 2)
    idx = ih * W + iw

    def j_step(i, jv):
        bit = lax.shift_left(jnp.int32(1), jnp.int32(14) - i)
        cand = jv | bit
        cnt = jnp.sum((eq & (idx < cand[:, None, None])).astype(jnp.int32),
                      axis=(1, 2))
        return jnp.where(cnt <= need, cand, jv)

    jv = lax.fori_loop(0, 15, j_step, jnp.zeros((N,), jnp.int32))

    sel = gt | (eq & (idx < jv[:, None, None]))
    row = lax.broadcasted_iota(jnp.int32, (N, H, W), 0)
    sel = sel | (row % Vb == 0)
    mask_ref[...] = sel.astype(jnp.float32)


def _apply_body(x_ref, mask_ref, xm_ref, fuse_ref):
    xm = x_ref[...] * mask_ref[:, None, :, :]
    xm_ref[...] = xm
    fuse_ref[...] = jnp.max(xm, axis=0, keepdims=True)


@jax.jit
def kernel(x, record_len, conf_map, lam, fc_w, fc_b):
    N, C, H, W = x.shape
    B = record_len.shape[0]
    Vb = N // B
    K = (H * W) // 2

    mask = pl.pallas_call(
        functools.partial(_mask_body, N=N, B=B, H=H, W=W, K=K),
        out_shape=jax.ShapeDtypeStruct((N, H, W), jnp.float32),
        in_specs=[
            pl.BlockSpec(memory_space=pltpu.VMEM),
            pl.BlockSpec(memory_space=pltpu.VMEM),
            pl.BlockSpec(memory_space=pltpu.VMEM),
            pl.BlockSpec(memory_space=pltpu.SMEM),
        ],
        out_specs=pl.BlockSpec(memory_space=pltpu.VMEM),
    )(conf_map, fc_w, fc_b, lam.reshape(1))

    CB = 8
    xm, x_fuse = pl.pallas_call(
        _apply_body,
        grid=(B, C // CB),
        in_specs=[
            pl.BlockSpec((Vb, CB, H, W), lambda b, c: (b, c, 0, 0)),
            pl.BlockSpec((Vb, H, W), lambda b, c: (b, 0, 0)),
        ],
        out_specs=[
            pl.BlockSpec((Vb, CB, H, W), lambda b, c: (b, c, 0, 0)),
            pl.BlockSpec((1, CB, H, W), lambda b, c: (b, c, 0, 0)),
        ],
        out_shape=[
            jax.ShapeDtypeStruct((N, C, H, W), jnp.float32),
            jax.ShapeDtypeStruct((B, C, H, W), jnp.float32),
        ],
        compiler_params=pltpu.CompilerParams(
            dimension_semantics=("parallel", "parallel"),
        ),
    )(x, mask)

    hw = jnp.float32(H * W)
    rates = jnp.float32(Vb * K) / (record_len.astype(jnp.float32) * hw)
    communication_rates = jnp.sum(rates) / jnp.float32(B)

    return (x_fuse, communication_rates, xm, jnp.float32(0.0))


# speedup vs baseline: 3.8603x; 3.8603x over previous
import functools

import jax
import jax.numpy as jnp
from jax import lax
from jax.experimental import pallas as pl
from jax.experimental.pallas import tpu as pltpu


def _mask_body(conf_ref, fcw_ref, fcb_ref, lam_ref, mask_ref, *, N, B, H, W, K):
    Vb = N // B
    conf = conf_ref[...]
    cm = jax.nn.sigmoid(conf).max(axis=1)
    lam = lam_ref[0]
    fcw = fcw_ref[...]
    fcb = fcb_ref[...]

    tf = (
        jnp.dot(cm.reshape(N * H, W), fcw.T, preferred_element_type=jnp.float32)
        + fcb[None, :]
    ).reshape(N, H, H)

    comms = []
    for v in range(N):
        ego = (v // Vb) * Vb
        req = 1.0 - cm[ego]
        diff = cm[v] * req
        sim = jnp.dot(tf[v], cm[v], preferred_element_type=jnp.float32)
        comms.append(lam * diff + (1.0 - lam) * sim)
    comm = jnp.stack(comms, axis=0)

    ki = lax.bitcast_convert_type(comm, jnp.int32)
    k3 = jnp.where(ki < 0, ki ^ jnp.int32(0x7FFFFFFF), ki)

    def t_step(i, tu):
        bit = lax.shift_left(jnp.int32(1), jnp.int32(31) - i)
        cand_u = tu | bit
        cand_s = cand_u ^ jnp.int32(-2147483648)
        cnt = jnp.sum((k3 >= cand_s[:, None, None]).astype(jnp.int32), axis=(1, 2))
        return jnp.where(cnt >= K, cand_u, tu)

    tu = lax.fori_loop(0, 32, t_step, jnp.zeros((N,), jnp.int32))
    ts = tu ^ jnp.int32(-2147483648)

    gt = k3 > ts[:, None, None]
    eq = k3 == ts[:, None, None]
    need = K - jnp.sum(gt.astype(jnp.int32), axis=(1, 2))

    ih = lax.broadcasted_iota(jnp.int32, (N, H, W), 1)
    iw = lax.broadcasted_iota(jnp.int32, (N, H, W), 2)
    idx = ih * W + iw

    def j_step(i, jv):
        bit = lax.shift_left(jnp.int32(1), jnp.int32(14) - i)
        cand = jv | bit
        cnt = jnp.sum((eq & (idx < cand[:, None, None])).astype(jnp.int32),
                      axis=(1, 2))
        return jnp.where(cnt <= need, cand, jv)

    jv = lax.fori_loop(0, 15, j_step, jnp.zeros((N,), jnp.int32))

    sel = gt | (eq & (idx < jv[:, None, None]))
    row = lax.broadcasted_iota(jnp.int32, (N, H, W), 0)
    sel = sel | (row % Vb == 0)
    mask_ref[...] = sel.astype(jnp.float32)


def _apply_body(x_ref, mask_ref, xm_ref, fuse_ref):
    xm = x_ref[...] * mask_ref[...][:, None, :, :]
    xm_ref[...] = xm
    fuse_ref[...] = jnp.max(xm, axis=0, keepdims=True)


@jax.jit
def kernel(x, record_len, conf_map, lam, fc_w, fc_b):
    N, C, H, W = x.shape
    B = record_len.shape[0]
    Vb = N // B
    K = (H * W) // 2

    mask = pl.pallas_call(
        functools.partial(_mask_body, N=N, B=B, H=H, W=W, K=K),
        out_shape=jax.ShapeDtypeStruct((N, H, W), jnp.float32),
        in_specs=[
            pl.BlockSpec(memory_space=pltpu.VMEM),
            pl.BlockSpec(memory_space=pltpu.VMEM),
            pl.BlockSpec(memory_space=pltpu.VMEM),
            pl.BlockSpec(memory_space=pltpu.SMEM),
        ],
        out_specs=pl.BlockSpec(memory_space=pltpu.VMEM),
    )(conf_map, fc_w, fc_b, lam.reshape(1))

    CB = 8
    xm, x_fuse = pl.pallas_call(
        _apply_body,
        grid=(B, C // CB),
        in_specs=[
            pl.BlockSpec((Vb, CB, H, W), lambda b, c: (b, c, 0, 0)),
            pl.BlockSpec((Vb, H, W), lambda b, c: (b, 0, 0)),
        ],
        out_specs=[
            pl.BlockSpec((Vb, CB, H, W), lambda b, c: (b, c, 0, 0)),
            pl.BlockSpec((1, CB, H, W), lambda b, c: (b, c, 0, 0)),
        ],
        out_shape=[
            jax.ShapeDtypeStruct((N, C, H, W), jnp.float32),
            jax.ShapeDtypeStruct((B, C, H, W), jnp.float32),
        ],
        compiler_params=pltpu.CompilerParams(
            dimension_semantics=("parallel", "parallel"),
        ),
    )(x, mask)

    hw = jnp.float32(H * W)
    rates = jnp.float32(Vb * K) / (record_len.astype(jnp.float32) * hw)
    communication_rates = jnp.sum(rates) / jnp.float32(B)

    return (x_fuse, communication_rates, xm, jnp.float32(0.0))
